# baseline (speedup 1.0000x reference)
import jax
import jax.numpy as jnp
from jax import lax
from jax.experimental import pallas as pl
from jax.experimental.pallas import tpu as pltpu
from jax.experimental.pallas import tpu_sc as plsc

_SPACING = (0.001, 0.001)
_N_KEYS = 2048
_N_QUERIES = 3 * _N_KEYS
_D = 128
_LANES = 16
_CHUNKS = _N_KEYS // _LANES
_NUM_CORES = 2
_NUM_SUBCORES = 16
_NUM_WORKERS = _NUM_CORES * _NUM_SUBCORES
_QPW = _N_QUERIES // _NUM_WORKERS
_GROUPS = _QPW // _LANES
_HGROUPS = _GROUPS // 2
_HQ = _QPW // 2
_NB = 1024
_BCHUNKS = _NB // _LANES
_BF = float(_NB)
_INV_BF = 1.0 / _BF


def _walk(skx_v, sky_v, qxv, qyv, p0, bd, bp):
    def cond(state):
        return jnp.any(state[4] | state[5])

    def body(state):
        pr, pl_, bd, bp, ar, al = state
        prc = jnp.clip(pr, 0, _N_KEYS - 1)
        plc = jnp.clip(pl_, 0, _N_KEYS - 1)
        kxr = plsc.load_gather(skx_v, [prc])
        kyr = plsc.load_gather(sky_v, [prc])
        kxl = plsc.load_gather(skx_v, [plc])
        kyl = plsc.load_gather(sky_v, [plc])
        dxr = qxv - kxr
        dyr = qyv - kyr
        dr = dxr * dxr + dyr * dyr
        dxl = qxv - kxl
        dyl = qyv - kyl
        dl = dxl * dxl + dyl * dyl
        vr = ar & (pr < _N_KEYS)
        vl = al & (pl_ >= 0)
        tr = vr & (dr < bd)
        bd = jnp.where(tr, dr, bd)
        bp = jnp.where(tr, pr, bp)
        tl = vl & (dl < bd)
        bd = jnp.where(tl, dl, bd)
        bp = jnp.where(tl, pl_, bp)
        bkr = (kxr * _BF).astype(jnp.int32)
        bkl = (kxl * _BF).astype(jnp.int32)
        gr = bkr.astype(jnp.float32) * _INV_BF - qxv
        gl = qxv - (bkl + 1).astype(jnp.float32) * _INV_BF
        nar = vr & ((gr <= 0.0) | (gr * gr <= bd))
        nal = vl & ((gl <= 0.0) | (gl * gl <= bd))
        return pr + 1, pl_ - 1, bd, bp, nar, nal

    on = jnp.full((_LANES,), True)
    _, _, bd, bp, _, _ = lax.while_loop(
        cond, body, (p0, p0 - 1, bd, bp, on, on)
    )
    return bd, bp


def _sc_body(x_hbm, y_hbm, values_hbm, out_hbm,
             x_v, y_v, skx_v, sky_v, sidx_v, inv_v, counts_v, bi_v, rows_v,
             sem):
    wid = lax.axis_index("s") * _NUM_CORES + lax.axis_index("c")
    base = wid * _QPW

    pltpu.sync_copy(x_hbm, x_v)
    pltpu.sync_copy(y_hbm, y_v)

    lane = lax.iota(jnp.int32, _LANES)
    zero = jnp.zeros((_LANES,), jnp.int32)

    def zero_body(c, _):
        counts_v[pl.ds(c * _LANES, _LANES)] = zero
        return 0

    lax.fori_loop(0, _BCHUNKS, zero_body, 0)

    def hist_body(c, _):
        xv = x_v[pl.ds(c * _LANES, _LANES)]
        b = (xv * _BF).astype(jnp.int32)
        cnt, last = plsc.scan_count(b)
        plsc.addupdate_scatter(counts_v, [b], cnt, mask=last)
        return 0

    lax.fori_loop(0, _CHUNKS, hist_body, 0)

    def scan_body(c, tot):
        s = counts_v[pl.ds(c * _LANES, _LANES)]
        cs = plsc.cumsum(s)
        counts_v[pl.ds(c * _LANES, _LANES)] = (tot + cs) - s
        return tot + cs[15]

    lax.fori_loop(0, _BCHUNKS, scan_body, jnp.int32(0))

    def place_body(c, _):
        xv = x_v[pl.ds(c * _LANES, _LANES)]
        yv = y_v[pl.ds(c * _LANES, _LANES)]
        b = (xv * _BF).astype(jnp.int32)
        cnt, last = plsc.scan_count(b)
        pos = plsc.load_gather(counts_v, [b]) + cnt - 1
        plsc.store_scatter(skx_v, [pos], xv)
        plsc.store_scatter(sky_v, [pos], yv)
        plsc.store_scatter(sidx_v, [pos], c * _LANES + lane)
        inv_v[pl.ds(c * _LANES, _LANES)] = pos
        plsc.addupdate_scatter(counts_v, [b], cnt, mask=last)
        return 0

    lax.fori_loop(0, _CHUNKS, place_body, 0)

    sx, sy = _SPACING
    half_x = jnp.float32(sx / 2.0)
    half_y = jnp.float32(sy / 2.0)

    def group_body(g, _):
        qidx = base + g * _LANES + lane
        src = qidx & (_N_KEYS - 1)
        copy_id = qidx >> 11
        xs = plsc.load_gather(x_v, [src])
        ys = plsc.load_gather(y_v, [src])
        qxv = jnp.where(copy_id == 0, xs - half_x, (xs + sx) - half_x)
        qyv = jnp.where(copy_id == 1, ys - half_y, (ys + sy) - half_y)
        p0 = plsc.load_gather(inv_v, [src])
        bd = jnp.full((_LANES,), jnp.inf, jnp.float32)
        bp = jnp.zeros((_LANES,), jnp.int32)
        bd, bp = _walk(skx_v, sky_v, qxv, qyv, p0, bd, bp)
        bi = plsc.load_gather(sidx_v, [bp])
        bi_v[g // _HGROUPS, pl.ds((g % _HGROUPS) * _LANES, _LANES)] = bi
        return 0

    lax.fori_loop(0, _GROUPS, group_body, 0)
    c0 = pltpu.async_copy(
        values_hbm.at[bi_v.at[0]], rows_v.at[pl.ds(0, _HQ)], sem
    )
    c1 = pltpu.async_copy(
        values_hbm.at[bi_v.at[1]], rows_v.at[pl.ds(_HQ, _HQ)], sem
    )
    c0.wait()
    c1.wait()
    pltpu.sync_copy(rows_v, out_hbm.at[pl.ds(base, _QPW)])


def _nn_gather(x, y, values_t):
    mesh = plsc.VectorSubcoreMesh(core_axis_name="c", subcore_axis_name="s")
    return pl.kernel(
        _sc_body,
        out_type=jax.ShapeDtypeStruct((_N_QUERIES, _D), jnp.float32),
        mesh=mesh,
        compiler_params=pltpu.CompilerParams(needs_layout_passes=False),
        scratch_types=[
            pltpu.VMEM((_N_KEYS,), jnp.float32),
            pltpu.VMEM((_N_KEYS,), jnp.float32),
            pltpu.VMEM((_N_KEYS,), jnp.float32),
            pltpu.VMEM((_N_KEYS,), jnp.float32),
            pltpu.VMEM((_N_KEYS,), jnp.int32),
            pltpu.VMEM((_N_KEYS,), jnp.int32),
            pltpu.VMEM((_NB,), jnp.int32),
            pltpu.VMEM((2, _HQ), jnp.int32),
            pltpu.VMEM((_QPW, _D), jnp.float32),
            pltpu.SemaphoreType.DMA,
        ],
    )(x, y, values_t)


def kernel(values, coords):
    sx, sy = _SPACING
    x = coords[:, 0]
    y = coords[:, 1]
    new_coords = jnp.concatenate(
        (
            jnp.stack((x, y + sy), axis=1),
            jnp.stack((x + sx, y), axis=1),
            jnp.stack((x + sx, y + sy), axis=1),
        ),
        axis=0,
    )
    sampled_coords = jnp.concatenate((coords, new_coords), axis=0)
    out_t = _nn_gather(x, y, values.T)
    out_values = jnp.concatenate((values, out_t.T), axis=1)
    return out_values, sampled_coords


# speedup vs baseline: 1.0562x; 1.0562x over previous
import jax
import jax.numpy as jnp
from jax import lax
from jax.experimental import pallas as pl
from jax.experimental.pallas import tpu as pltpu
from jax.experimental.pallas import tpu_sc as plsc

_SPACING = (0.001, 0.001)
_N_KEYS = 2048
_N_QUERIES = 3 * _N_KEYS
_D = 128
_LANES = 16
_CHUNKS = _N_KEYS // _LANES
_NUM_CORES = 2
_NUM_SUBCORES = 16
_NUM_WORKERS = _NUM_CORES * _NUM_SUBCORES
_QPW = _N_QUERIES // _NUM_WORKERS
_GROUPS = _QPW // _LANES
_HGROUPS = _GROUPS // 2
_HQ = _QPW // 2
_NB = 1024
_BCHUNKS = _NB // _LANES
_BF = float(_NB)
_INV_BF = 1.0 / _BF


def _walk(skx_v, sky_v, qxv, qyv, p0, bd, bp):
    def cond(state):
        return jnp.any(state[4] | state[5])

    def body(state):
        pr, pl_, bd, bp, ar, al = state
        vr = ar & (pr < _N_KEYS)
        vl = al & (pl_ >= 0)
        kxr = plsc.load_gather(skx_v, [pr], mask=vr)
        kyr = plsc.load_gather(sky_v, [pr], mask=vr)
        kxl = plsc.load_gather(skx_v, [pl_], mask=vl)
        kyl = plsc.load_gather(sky_v, [pl_], mask=vl)
        dxr = qxv - kxr
        dyr = qyv - kyr
        dr = dxr * dxr + dyr * dyr
        dxl = qxv - kxl
        dyl = qyv - kyl
        dl = dxl * dxl + dyl * dyl
        tr = vr & (dr < bd)
        bd = jnp.where(tr, dr, bd)
        bp = jnp.where(tr, pr, bp)
        tl = vl & (dl < bd)
        bd = jnp.where(tl, dl, bd)
        bp = jnp.where(tl, pl_, bp)
        bkr = (kxr * _BF).astype(jnp.int32)
        bkl = (kxl * _BF).astype(jnp.int32)
        gr = bkr.astype(jnp.float32) * _INV_BF - qxv
        gl = qxv - (bkl + 1).astype(jnp.float32) * _INV_BF
        nar = vr & ((gr <= 0.0) | (gr * gr <= bd))
        nal = vl & ((gl <= 0.0) | (gl * gl <= bd))
        return pr + 1, pl_ - 1, bd, bp, nar, nal

    on = jnp.full((_LANES,), True)
    _, _, bd, bp, _, _ = lax.while_loop(
        cond, body, (p0 + 1, p0 - 1, bd, bp, on, on)
    )
    return bd, bp


def _sc_body(x_hbm, y_hbm, values_hbm, out_hbm,
             x_v, y_v, skx_v, sky_v, sidx_v, inv_v, counts_v, bi_v, rows_v,
             sem):
    wid = lax.axis_index("s") * _NUM_CORES + lax.axis_index("c")
    base = wid * _QPW

    pltpu.sync_copy(x_hbm, x_v)
    pltpu.sync_copy(y_hbm, y_v)

    lane = lax.iota(jnp.int32, _LANES)
    zero = jnp.zeros((_LANES,), jnp.int32)

    @plsc.parallel_loop(0, _NB, _LANES)
    def zero_body(i):
        counts_v[pl.ds(i, _LANES)] = zero

    @plsc.parallel_loop(0, _N_KEYS, _LANES, unroll=2)
    def hist_body(i):
        xv = x_v[pl.ds(i, _LANES)]
        b = (xv * _BF).astype(jnp.int32)
        cnt, last = plsc.scan_count(b)
        plsc.addupdate_scatter(counts_v, [b], cnt, mask=last)

    def scan_body(c, tot):
        s = counts_v[pl.ds(c * _LANES, _LANES)]
        cs = plsc.cumsum(s)
        counts_v[pl.ds(c * _LANES, _LANES)] = (tot + cs) - s
        return tot + cs[15]

    lax.fori_loop(0, _BCHUNKS, scan_body, jnp.int32(0))

    def place_body(c, _):
        xv = x_v[pl.ds(c * _LANES, _LANES)]
        yv = y_v[pl.ds(c * _LANES, _LANES)]
        b = (xv * _BF).astype(jnp.int32)
        cnt, last = plsc.scan_count(b)
        pos = plsc.load_gather(counts_v, [b]) + cnt - 1
        plsc.store_scatter(skx_v, [pos], xv)
        plsc.store_scatter(sky_v, [pos], yv)
        plsc.store_scatter(sidx_v, [pos], c * _LANES + lane)
        inv_v[pl.ds(c * _LANES, _LANES)] = pos
        plsc.addupdate_scatter(counts_v, [b], cnt, mask=last)
        return 0

    lax.fori_loop(0, _CHUNKS, place_body, 0)

    sx, sy = _SPACING
    half_x = jnp.float32(sx / 2.0)
    half_y = jnp.float32(sy / 2.0)

    def group_body(g, _):
        qidx = base + g * _LANES + lane
        src = qidx & (_N_KEYS - 1)
        copy_id = qidx >> 11
        xs = plsc.load_gather(x_v, [src])
        ys = plsc.load_gather(y_v, [src])
        qxv = jnp.where(copy_id == 0, xs - half_x, (xs + sx) - half_x)
        qyv = jnp.where(copy_id == 1, ys - half_y, (ys + sy) - half_y)
        p0 = plsc.load_gather(inv_v, [src])
        dx0 = qxv - xs
        dy0 = qyv - ys
        bd = dx0 * dx0 + dy0 * dy0
        bp = p0
        bd, bp = _walk(skx_v, sky_v, qxv, qyv, p0, bd, bp)
        bi = plsc.load_gather(sidx_v, [bp])
        bi_v[g // _HGROUPS, pl.ds((g % _HGROUPS) * _LANES, _LANES)] = bi
        return 0

    lax.fori_loop(0, _GROUPS, group_body, 0)
    c0 = pltpu.async_copy(
        values_hbm.at[bi_v.at[0]], rows_v.at[pl.ds(0, _HQ)], sem
    )
    c1 = pltpu.async_copy(
        values_hbm.at[bi_v.at[1]], rows_v.at[pl.ds(_HQ, _HQ)], sem
    )
    c0.wait()
    c1.wait()
    pltpu.sync_copy(rows_v, out_hbm.at[pl.ds(base, _QPW)])


def _nn_gather(x, y, values_t):
    mesh = plsc.VectorSubcoreMesh(core_axis_name="c", subcore_axis_name="s")
    return pl.kernel(
        _sc_body,
        out_type=jax.ShapeDtypeStruct((_N_QUERIES, _D), jnp.float32),
        mesh=mesh,
        compiler_params=pltpu.CompilerParams(needs_layout_passes=False),
        scratch_types=[
            pltpu.VMEM((_N_KEYS,), jnp.float32),
            pltpu.VMEM((_N_KEYS,), jnp.float32),
            pltpu.VMEM((_N_KEYS,), jnp.float32),
            pltpu.VMEM((_N_KEYS,), jnp.float32),
            pltpu.VMEM((_N_KEYS,), jnp.int32),
            pltpu.VMEM((_N_KEYS,), jnp.int32),
            pltpu.VMEM((_NB,), jnp.int32),
            pltpu.VMEM((2, _HQ), jnp.int32),
            pltpu.VMEM((_QPW, _D), jnp.float32),
            pltpu.SemaphoreType.DMA,
        ],
    )(x, y, values_t)


def kernel(values, coords):
    sx, sy = _SPACING
    x = coords[:, 0]
    y = coords[:, 1]
    new_coords = jnp.concatenate(
        (
            jnp.stack((x, y + sy), axis=1),
            jnp.stack((x + sx, y), axis=1),
            jnp.stack((x + sx, y + sy), axis=1),
        ),
        axis=0,
    )
    sampled_coords = jnp.concatenate((coords, new_coords), axis=0)
    out_t = _nn_gather(x, y, values.T)
    out_values = jnp.concatenate((values, out_t.T), axis=1)
    return out_values, sampled_coords


# speedup vs baseline: 1.0659x; 1.0091x over previous
import jax
import jax.numpy as jnp
from jax import lax
from jax.experimental import pallas as pl
from jax.experimental.pallas import tpu as pltpu
from jax.experimental.pallas import tpu_sc as plsc

_SPACING = (0.001, 0.001)
_N_KEYS = 2048
_N_QUERIES = 3 * _N_KEYS
_D = 128
_LANES = 16
_CHUNKS = _N_KEYS // _LANES
_NUM_CORES = 2
_NUM_SUBCORES = 16
_NUM_WORKERS = _NUM_CORES * _NUM_SUBCORES
_QPW = _N_QUERIES // _NUM_WORKERS
_GROUPS = _QPW // _LANES
_HGROUPS = _GROUPS // 2
_HQ = _QPW // 2
_NB = 1024
_BCHUNKS = _NB // _LANES
_BF = float(_NB)
_INV_BF = 1.0 / _BF


def _walk(skx_v, sky_v, qxv, qyv, p0, bd, bp):
    def cond(state):
        return jnp.any(state[4] | state[5])

    def body(state):
        pr, pl_, bd, bp, ar, al = state
        vr = ar & (pr < _N_KEYS)
        vl = al & (pl_ >= 0)
        kxr = plsc.load_gather(skx_v, [pr], mask=vr)
        kyr = plsc.load_gather(sky_v, [pr], mask=vr)
        kxl = plsc.load_gather(skx_v, [pl_], mask=vl)
        kyl = plsc.load_gather(sky_v, [pl_], mask=vl)
        dxr = qxv - kxr
        dyr = qyv - kyr
        dr = dxr * dxr + dyr * dyr
        dxl = qxv - kxl
        dyl = qyv - kyl
        dl = dxl * dxl + dyl * dyl
        tr = vr & (dr < bd)
        bd = jnp.where(tr, dr, bd)
        bp = jnp.where(tr, pr, bp)
        tl = vl & (dl < bd)
        bd = jnp.where(tl, dl, bd)
        bp = jnp.where(tl, pl_, bp)
        bkr = (kxr * _BF).astype(jnp.int32)
        bkl = (kxl * _BF).astype(jnp.int32)
        gr = bkr.astype(jnp.float32) * _INV_BF - qxv
        gl = qxv - (bkl + 1).astype(jnp.float32) * _INV_BF
        nar = vr & ((gr <= 0.0) | (gr * gr <= bd))
        nal = vl & ((gl <= 0.0) | (gl * gl <= bd))
        return pr + 1, pl_ - 1, bd, bp, nar, nal

    on = jnp.full((_LANES,), True)
    _, _, bd, bp, _, _ = lax.while_loop(
        cond, body, (p0 + 1, p0 - 1, bd, bp, on, on)
    )
    return bd, bp


def _sc_body(x_hbm, y_hbm, values_hbm, out_hbm,
             x_v, y_v, skx_v, sky_v, sidx_v, inv_v, counts_v, bi_v, rows_v,
             sem):
    wid = lax.axis_index("s") * _NUM_CORES + lax.axis_index("c")
    base = wid * _QPW

    pltpu.sync_copy(x_hbm, x_v)
    pltpu.sync_copy(y_hbm, y_v)

    lane = lax.iota(jnp.int32, _LANES)
    zero = jnp.zeros((_LANES,), jnp.int32)

    @plsc.parallel_loop(0, _NB, _LANES)
    def zero_body(i):
        counts_v[pl.ds(i, _LANES)] = zero

    @plsc.parallel_loop(0, _N_KEYS, _LANES, unroll=2)
    def hist_body(i):
        xv = x_v[pl.ds(i, _LANES)]
        b = (xv * _BF).astype(jnp.int32)
        cnt, last = plsc.scan_count(b)
        plsc.addupdate_scatter(counts_v, [b], cnt, mask=last)

    @plsc.parallel_loop(0, _NB, _LANES, carry=jnp.int32(0))
    def scan_body(i, tot):
        s = counts_v[pl.ds(i, _LANES)]
        cs = plsc.cumsum(s)
        counts_v[pl.ds(i, _LANES)] = (tot + cs) - s
        return tot + cs[15]

    def place_pair(c, _):
        for h in range(2):
            off = (c * 2 + h) * _LANES
            xv = x_v[pl.ds(off, _LANES)]
            yv = y_v[pl.ds(off, _LANES)]
            b = (xv * _BF).astype(jnp.int32)
            cnt, last = plsc.scan_count(b)
            pos = plsc.load_gather(counts_v, [b]) + cnt - 1
            plsc.store_scatter(skx_v, [pos], xv)
            plsc.store_scatter(sky_v, [pos], yv)
            plsc.store_scatter(sidx_v, [pos], off + lane)
            inv_v[pl.ds(off, _LANES)] = pos
            plsc.addupdate_scatter(counts_v, [b], cnt, mask=last)
        return 0

    lax.fori_loop(0, _CHUNKS // 2, place_pair, 0)

    sx, sy = _SPACING
    half_x = jnp.float32(sx / 2.0)
    half_y = jnp.float32(sy / 2.0)

    def group_body(g, _):
        qidx = base + g * _LANES + lane
        src = qidx & (_N_KEYS - 1)
        copy_id = qidx >> 11
        xs = plsc.load_gather(x_v, [src])
        ys = plsc.load_gather(y_v, [src])
        qxv = jnp.where(copy_id == 0, xs - half_x, (xs + sx) - half_x)
        qyv = jnp.where(copy_id == 1, ys - half_y, (ys + sy) - half_y)
        p0 = plsc.load_gather(inv_v, [src])
        dx0 = qxv - xs
        dy0 = qyv - ys
        bd = dx0 * dx0 + dy0 * dy0
        bp = p0
        bd, bp = _walk(skx_v, sky_v, qxv, qyv, p0, bd, bp)
        bi = plsc.load_gather(sidx_v, [bp])
        bi_v[g // _HGROUPS, pl.ds((g % _HGROUPS) * _LANES, _LANES)] = bi
        return 0

    lax.fori_loop(0, _GROUPS, group_body, 0)
    c0 = pltpu.async_copy(
        values_hbm.at[bi_v.at[0]], rows_v.at[pl.ds(0, _HQ)], sem
    )
    c1 = pltpu.async_copy(
        values_hbm.at[bi_v.at[1]], rows_v.at[pl.ds(_HQ, _HQ)], sem
    )
    c0.wait()
    c1.wait()
    pltpu.sync_copy(rows_v, out_hbm.at[pl.ds(base, _QPW)])


def _nn_gather(x, y, values_t):
    mesh = plsc.VectorSubcoreMesh(core_axis_name="c", subcore_axis_name="s")
    return pl.kernel(
        _sc_body,
        out_type=jax.ShapeDtypeStruct((_N_QUERIES, _D), jnp.float32),
        mesh=mesh,
        compiler_params=pltpu.CompilerParams(needs_layout_passes=False),
        scratch_types=[
            pltpu.VMEM((_N_KEYS,), jnp.float32),
            pltpu.VMEM((_N_KEYS,), jnp.float32),
            pltpu.VMEM((_N_KEYS,), jnp.float32),
            pltpu.VMEM((_N_KEYS,), jnp.float32),
            pltpu.VMEM((_N_KEYS,), jnp.int32),
            pltpu.VMEM((_N_KEYS,), jnp.int32),
            pltpu.VMEM((_NB,), jnp.int32),
            pltpu.VMEM((2, _HQ), jnp.int32),
            pltpu.VMEM((_QPW, _D), jnp.float32),
            pltpu.SemaphoreType.DMA,
        ],
    )(x, y, values_t)


def kernel(values, coords):
    sx, sy = _SPACING
    x = coords[:, 0]
    y = coords[:, 1]
    new_coords = jnp.concatenate(
        (
            jnp.stack((x, y + sy), axis=1),
            jnp.stack((x + sx, y), axis=1),
            jnp.stack((x + sx, y + sy), axis=1),
        ),
        axis=0,
    )
    sampled_coords = jnp.concatenate((coords, new_coords), axis=0)
    out_t = _nn_gather(x, y, values.T)
    out_values = jnp.concatenate((values, out_t.T), axis=1)
    return out_values, sampled_coords


# speedup vs baseline: 1.0662x; 1.0003x over previous
import jax
import jax.numpy as jnp
from jax import lax
from jax.experimental import pallas as pl
from jax.experimental.pallas import tpu as pltpu
from jax.experimental.pallas import tpu_sc as plsc

_SPACING = (0.001, 0.001)
_N_KEYS = 2048
_N_QUERIES = 3 * _N_KEYS
_D = 128
_LANES = 16
_CHUNKS = _N_KEYS // _LANES
_NUM_CORES = 2
_NUM_SUBCORES = 16
_NUM_WORKERS = _NUM_CORES * _NUM_SUBCORES
_QPW = _N_QUERIES // _NUM_WORKERS
_GROUPS = _QPW // _LANES
_HGROUPS = _GROUPS // 2
_HQ = _QPW // 2
_NB = 1024
_BCHUNKS = _NB // _LANES
_BF = float(_NB)
_INV_BF = 1.0 / _BF


def _walk(skx_v, sky_v, qxv, qyv, p0, bd, bp):
    def cond(state):
        return jnp.any(state[4] | state[5])

    def body(state):
        pr, pl_, bd, bp, ar, al = state
        vr = ar & (pr < _N_KEYS)
        vl = al & (pl_ >= 0)
        kxr = plsc.load_gather(skx_v, [pr], mask=vr)
        kyr = plsc.load_gather(sky_v, [pr], mask=vr)
        kxl = plsc.load_gather(skx_v, [pl_], mask=vl)
        kyl = plsc.load_gather(sky_v, [pl_], mask=vl)
        dxr = qxv - kxr
        dyr = qyv - kyr
        dr = dxr * dxr + dyr * dyr
        dxl = qxv - kxl
        dyl = qyv - kyl
        dl = dxl * dxl + dyl * dyl
        tr = vr & (dr < bd)
        bd = jnp.where(tr, dr, bd)
        bp = jnp.where(tr, pr, bp)
        tl = vl & (dl < bd)
        bd = jnp.where(tl, dl, bd)
        bp = jnp.where(tl, pl_, bp)
        bkr = (kxr * _BF).astype(jnp.int32)
        bkl = (kxl * _BF).astype(jnp.int32)
        gr = bkr.astype(jnp.float32) * _INV_BF - qxv
        gl = qxv - (bkl + 1).astype(jnp.float32) * _INV_BF
        nar = vr & ((gr <= 0.0) | (gr * gr <= bd))
        nal = vl & ((gl <= 0.0) | (gl * gl <= bd))
        return pr + 1, pl_ - 1, bd, bp, nar, nal

    on = jnp.full((_LANES,), True)
    _, _, bd, bp, _, _ = lax.while_loop(
        cond, body, (p0 + 1, p0 - 1, bd, bp, on, on)
    )
    return bd, bp


def _sc_body(x_hbm, y_hbm, values_hbm, out_hbm,
             x_v, y_v, skx_v, sky_v, sidx_v, inv_v, counts_v, bi_v, rows_v,
             sem):
    wid = lax.axis_index("s") * _NUM_CORES + lax.axis_index("c")
    base = wid * _QPW

    pltpu.sync_copy(x_hbm, x_v)
    pltpu.sync_copy(y_hbm, y_v)

    lane = lax.iota(jnp.int32, _LANES)
    zero = jnp.zeros((_LANES,), jnp.int32)

    @plsc.parallel_loop(0, _NB, _LANES)
    def zero_body(i):
        counts_v[pl.ds(i, _LANES)] = zero

    @plsc.parallel_loop(0, _N_KEYS, _LANES, unroll=4)
    def hist_body(i):
        xv = x_v[pl.ds(i, _LANES)]
        b = (xv * _BF).astype(jnp.int32)
        cnt, last = plsc.scan_count(b)
        plsc.addupdate_scatter(counts_v, [b], cnt, mask=last)

    @plsc.parallel_loop(0, _NB, _LANES, carry=jnp.int32(0))
    def scan_body(i, tot):
        s = counts_v[pl.ds(i, _LANES)]
        cs = plsc.cumsum(s)
        counts_v[pl.ds(i, _LANES)] = (tot + cs) - s
        return tot + cs[15]

    def place_pair(c, _):
        for h in range(2):
            off = (c * 2 + h) * _LANES
            xv = x_v[pl.ds(off, _LANES)]
            yv = y_v[pl.ds(off, _LANES)]
            b = (xv * _BF).astype(jnp.int32)
            cnt, last = plsc.scan_count(b)
            pos = plsc.load_gather(counts_v, [b]) + cnt - 1
            plsc.store_scatter(skx_v, [pos], xv)
            plsc.store_scatter(sky_v, [pos], yv)
            plsc.store_scatter(sidx_v, [pos], off + lane)
            inv_v[pl.ds(off, _LANES)] = pos
            plsc.addupdate_scatter(counts_v, [b], cnt, mask=last)
        return 0

    lax.fori_loop(0, _CHUNKS // 2, place_pair, 0)

    sx, sy = _SPACING
    half_x = jnp.float32(sx / 2.0)
    half_y = jnp.float32(sy / 2.0)

    def group_body(g, _):
        qidx = base + g * _LANES + lane
        src = qidx & (_N_KEYS - 1)
        copy_id = qidx >> 11
        xs = plsc.load_gather(x_v, [src])
        ys = plsc.load_gather(y_v, [src])
        qxv = jnp.where(copy_id == 0, xs - half_x, (xs + sx) - half_x)
        qyv = jnp.where(copy_id == 1, ys - half_y, (ys + sy) - half_y)
        p0 = plsc.load_gather(inv_v, [src])
        dx0 = qxv - xs
        dy0 = qyv - ys
        bd = dx0 * dx0 + dy0 * dy0
        bp = p0
        bd, bp = _walk(skx_v, sky_v, qxv, qyv, p0, bd, bp)
        bi = plsc.load_gather(sidx_v, [bp])
        bi_v[g // _HGROUPS, pl.ds((g % _HGROUPS) * _LANES, _LANES)] = bi
        return 0

    lax.fori_loop(0, _GROUPS, group_body, 0, unroll=2)
    c0 = pltpu.async_copy(
        values_hbm.at[bi_v.at[0]], rows_v.at[pl.ds(0, _HQ)], sem
    )
    c1 = pltpu.async_copy(
        values_hbm.at[bi_v.at[1]], rows_v.at[pl.ds(_HQ, _HQ)], sem
    )
    c0.wait()
    c1.wait()
    pltpu.sync_copy(rows_v, out_hbm.at[pl.ds(base, _QPW)])


def _nn_gather(x, y, values_t):
    mesh = plsc.VectorSubcoreMesh(core_axis_name="c", subcore_axis_name="s")
    return pl.kernel(
        _sc_body,
        out_type=jax.ShapeDtypeStruct((_N_QUERIES, _D), jnp.float32),
        mesh=mesh,
        compiler_params=pltpu.CompilerParams(needs_layout_passes=False),
        scratch_types=[
            pltpu.VMEM((_N_KEYS,), jnp.float32),
            pltpu.VMEM((_N_KEYS,), jnp.float32),
            pltpu.VMEM((_N_KEYS,), jnp.float32),
            pltpu.VMEM((_N_KEYS,), jnp.float32),
            pltpu.VMEM((_N_KEYS,), jnp.int32),
            pltpu.VMEM((_N_KEYS,), jnp.int32),
            pltpu.VMEM((_NB,), jnp.int32),
            pltpu.VMEM((2, _HQ), jnp.int32),
            pltpu.VMEM((_QPW, _D), jnp.float32),
            pltpu.SemaphoreType.DMA,
        ],
    )(x, y, values_t)


def kernel(values, coords):
    sx, sy = _SPACING
    x = coords[:, 0]
    y = coords[:, 1]
    new_coords = jnp.concatenate(
        (
            jnp.stack((x, y + sy), axis=1),
            jnp.stack((x + sx, y), axis=1),
            jnp.stack((x + sx, y + sy), axis=1),
        ),
        axis=0,
    )
    sampled_coords = jnp.concatenate((coords, new_coords), axis=0)
    out_t = _nn_gather(x, y, values.T)
    out_values = jnp.concatenate((values, out_t.T), axis=1)
    return out_values, sampled_coords
